# baseline (speedup 1.0000x reference)
import functools

import jax
import jax.numpy as jnp
from jax import lax
from jax.experimental import pallas as pl
from jax.experimental.pallas import tpu as pltpu
from jax.experimental.pallas import tpu_sc as plsc

_D = 64
_NC = 2
_NS = 16
_NW = _NC * _NS
_K = 16
_CH = _K * 16
_NBUF = 4
_LEAD = 2
_SCALE = 8.0


def _tec_body(n_chunks, idx_hbm, table_hbm, out_hbm, idx_v, bufs, gsems,
              ssems):
    wid = lax.axis_index("s") * _NC + lax.axis_index("c")
    nper = n_chunks * _CH
    pltpu.sync_copy(idx_hbm.at[pl.ds(wid * nper, nper)], idx_v)
    base = wid * nper

    def gather_start(j, b):
        for k in range(_K):
            ids = idx_v[pl.ds(j * _CH + k * 16, 16)] * 2
            pltpu.async_copy(table_hbm.at[ids], bufs[b].at[pl.ds(k * 16, 16)],
                             gsems[b])

    def gather_wait(b):
        pltpu.make_async_copy(table_hbm.at[pl.ds(0, _CH)], bufs[b],
                              gsems[b]).wait()

    def scatter_start(j, b):
        pltpu.async_copy(bufs[b], out_hbm.at[pl.ds(base + j * _CH, _CH)],
                         ssems[b])

    def scatter_wait(j, b):
        pltpu.make_async_copy(bufs[b], out_hbm.at[pl.ds(base + j * _CH, _CH)],
                              ssems[b]).wait()

    for j in range(_LEAD):
        gather_start(j, j % _NBUF)

    def outer(g):
        for b in range(_NBUF):
            j = g + b
            gather_wait(b)
            scatter_start(j, b)
            jn = j + _LEAD
            bn = (b + _LEAD) % _NBUF

            @pl.when(jn < n_chunks)
            def _():
                @pl.when(jn >= _NBUF)
                def _():
                    scatter_wait(jn - _NBUF, bn)

                gather_start(jn, bn)

    pl.loop(0, n_chunks, step=_NBUF)(outer)

    for b in range(_NBUF):
        scatter_wait(n_chunks - _NBUF + b, b)


@jax.jit
def kernel(x, table):
    b, h = x.shape
    n = b * h
    v, d = table.shape
    assert d == _D and n % (_NW * _CH * _NBUF) == 0
    n_chunks = n // (_NW * _CH)
    idx = x.reshape(n).astype(jnp.int32)
    tab = jnp.pad(table * jnp.float32(_SCALE),
                  ((0, 0), (0, _D))).reshape(2 * v, _D)

    mesh = plsc.VectorSubcoreMesh(core_axis_name="c", subcore_axis_name="s")
    rows = pl.kernel(
        functools.partial(_tec_body, n_chunks),
        out_type=jax.ShapeDtypeStruct((n, _D), jnp.float32),
        mesh=mesh,
        scratch_types=[
            pltpu.VMEM((n_chunks * _CH,), jnp.int32),
            [pltpu.VMEM((_CH, _D), jnp.float32) for _ in range(_NBUF)],
            [pltpu.SemaphoreType.DMA for _ in range(_NBUF)],
            [pltpu.SemaphoreType.DMA for _ in range(_NBUF)],
        ],
        compiler_params=pltpu.CompilerParams(use_tc_tiling_on_sc=False),
    )(idx, tab)
    return rows.reshape(b, h, _D)


# speedup vs baseline: 1.2995x; 1.2995x over previous
import functools

import jax
import jax.numpy as jnp
from jax import lax
from jax.experimental import pallas as pl
from jax.experimental.pallas import tpu as pltpu
from jax.experimental.pallas import tpu_sc as plsc

_D = 64
_NC = 2
_NS = 16
_NW = _NC * _NS
_K = 8
_CH = _K * 16
_NBUF = 4
_LEAD = 2
_SCALE = 8.0


def _tec_body(n_chunks, idx_hbm, table_hbm, out_hbm, idx_v, gbufs, sbufs,
              gsems, ssems):
    wid = lax.axis_index("s") * _NC + lax.axis_index("c")
    nper = n_chunks * _CH
    pltpu.sync_copy(idx_hbm.at[pl.ds(wid * nper, nper)], idx_v)
    base = wid * nper

    def gather_start(j, b):
        for k in range(_K):
            ids = idx_v[pl.ds(j * _CH + k * 16, 16)]
            pltpu.async_copy(table_hbm.at[ids], gbufs[b].at[pl.ds(k * 16, 16)],
                             gsems[b])

    def gather_wait(b):
        pltpu.make_async_copy(table_hbm.at[pl.ds(0, _CH)], gbufs[b],
                              gsems[b]).wait()

    def scatter_start(j, b):
        pltpu.async_copy(sbufs[b], out_hbm.at[pl.ds(base + j * _CH, _CH)],
                         ssems[b])

    def scatter_wait(j, b):
        pltpu.make_async_copy(sbufs[b], out_hbm.at[pl.ds(base + j * _CH, _CH)],
                              ssems[b]).wait()

    for j in range(_LEAD):
        gather_start(j, j % _NBUF)

    def outer(g):
        for b in range(_NBUF):
            j = g + b
            gather_wait(b)

            @pl.when(j >= _NBUF)
            def _():
                scatter_wait(j - _NBUF, b)

            gbuf, sbuf = gbufs[b], sbufs[b]

            def row(i):
                for c in range(_D // 16):
                    sl = pl.ds(c * 16, 16)
                    sbuf[i, sl] = gbuf[i, sl] * _SCALE

            plsc.parallel_loop(0, _CH, 1, unroll=4)(row)

            scatter_start(j, b)
            jn = j + _LEAD
            bn = (b + _LEAD) % _NBUF

            @pl.when(jn < n_chunks)
            def _():
                gather_start(jn, bn)

    pl.loop(0, n_chunks, step=_NBUF)(outer)

    for b in range(_NBUF):
        scatter_wait(n_chunks - _NBUF + b, b)


@jax.jit
def kernel(x, table):
    b, h = x.shape
    n = b * h
    assert n % (_NW * _CH * _NBUF) == 0
    n_chunks = n // (_NW * _CH)
    idx = x.reshape(n).astype(jnp.int32)

    mesh = plsc.VectorSubcoreMesh(core_axis_name="c", subcore_axis_name="s")
    out_padded = pl.kernel(
        functools.partial(_tec_body, n_chunks),
        out_type=jax.ShapeDtypeStruct((n, 2 * _D), jnp.float32),
        mesh=mesh,
        scratch_types=[
            pltpu.VMEM((n_chunks * _CH,), jnp.int32),
            [pltpu.VMEM((_CH, _D), jnp.float32) for _ in range(_NBUF)],
            [pltpu.VMEM((_CH, 2 * _D), jnp.float32) for _ in range(_NBUF)],
            [pltpu.SemaphoreType.DMA for _ in range(_NBUF)],
            [pltpu.SemaphoreType.DMA for _ in range(_NBUF)],
        ],
        compiler_params=pltpu.CompilerParams(use_tc_tiling_on_sc=False),
    )(idx, table)
    return out_padded[:, :_D].reshape(b, h, _D)


# speedup vs baseline: 1.3942x; 1.0729x over previous
import functools

import jax
import jax.numpy as jnp
from jax import lax
from jax.experimental import pallas as pl
from jax.experimental.pallas import tpu as pltpu
from jax.experimental.pallas import tpu_sc as plsc

_D = 64
_NC = 2
_NS = 16
_NW = _NC * _NS
_K = 8
_CH = _K * 16
_NBUF = 4
_LEAD = 2
_SCALE = 8.0


def _tec_body(n_chunks, idx_hbm, table_hbm, out_hbm, idx_v, bufs, gsems,
              ssems):
    wid = lax.axis_index("s") * _NC + lax.axis_index("c")
    nper = n_chunks * _CH
    pltpu.sync_copy(idx_hbm.at[pl.ds(wid * nper, nper)], idx_v)
    base = wid * nper

    def gather_start(j, b):
        for k in range(_K):
            ids = idx_v[pl.ds(j * _CH + k * 16, 16)]
            pltpu.async_copy(table_hbm.at[ids], bufs[b].at[pl.ds(k * 16, 16)],
                             gsems[b])

    def gather_wait(b):
        pltpu.make_async_copy(table_hbm.at[pl.ds(0, _CH)], bufs[b],
                              gsems[b]).wait()

    def out_window(j):
        return out_hbm.at[pl.ds(base + j * _CH, _CH), pl.ds(0, _D)]

    def scatter_start(j, b):
        pltpu.async_copy(bufs[b], out_window(j), ssems[b])

    def scatter_wait(j, b):
        pltpu.make_async_copy(bufs[b], out_window(j), ssems[b]).wait()

    for j in range(_LEAD):
        gather_start(j, j % _NBUF)

    def outer(g):
        for b in range(_NBUF):
            j = g + b
            gather_wait(b)
            buf = bufs[b]

            def row(i):
                for c in range(_D // 16):
                    sl = pl.ds(c * 16, 16)
                    buf[i, sl] = buf[i, sl] * _SCALE

            plsc.parallel_loop(0, _CH, 1, unroll=4)(row)

            scatter_start(j, b)
            jn = j + _LEAD
            bn = (b + _LEAD) % _NBUF

            @pl.when(jn < n_chunks)
            def _():
                @pl.when(jn >= _NBUF)
                def _():
                    scatter_wait(jn - _NBUF, bn)

                gather_start(jn, bn)

    pl.loop(0, n_chunks, step=_NBUF)(outer)

    for b in range(_NBUF):
        scatter_wait(n_chunks - _NBUF + b, b)


@jax.jit
def kernel(x, table):
    b, h = x.shape
    n = b * h
    assert n % (_NW * _CH * _NBUF) == 0
    n_chunks = n // (_NW * _CH)
    idx = x.reshape(n).astype(jnp.int32)

    mesh = plsc.VectorSubcoreMesh(core_axis_name="c", subcore_axis_name="s")
    out_padded = pl.kernel(
        functools.partial(_tec_body, n_chunks),
        out_type=jax.ShapeDtypeStruct((n, 2 * _D), jnp.float32),
        mesh=mesh,
        scratch_types=[
            pltpu.VMEM((n_chunks * _CH,), jnp.int32),
            [pltpu.VMEM((_CH, _D), jnp.float32) for _ in range(_NBUF)],
            [pltpu.SemaphoreType.DMA for _ in range(_NBUF)],
            [pltpu.SemaphoreType.DMA for _ in range(_NBUF)],
        ],
        compiler_params=pltpu.CompilerParams(use_tc_tiling_on_sc=False),
    )(idx, table)
    return out_padded[:, :_D].reshape(b, h, _D)


# speedup vs baseline: 1.4145x; 1.0146x over previous
import functools

import jax
import jax.numpy as jnp
from jax import lax
from jax.experimental import pallas as pl
from jax.experimental.pallas import tpu as pltpu
from jax.experimental.pallas import tpu_sc as plsc

_D = 64
_NC = 2
_NS = 16
_NW = _NC * _NS
_K = 16
_CH = _K * 16
_NBUF = 4
_LEAD = 2
_SCALE = 8.0


def _tec_body(n_chunks, idx_hbm, table_hbm, out_hbm, idx_v, bufs, gsems,
              ssems):
    wid = lax.axis_index("s") * _NC + lax.axis_index("c")
    nper = n_chunks * _CH
    pltpu.sync_copy(idx_hbm.at[pl.ds(wid * nper, nper)], idx_v)
    base = wid * nper

    def gather_start(j, b):
        for k in range(_K):
            ids = idx_v[pl.ds(j * _CH + k * 16, 16)]
            pltpu.async_copy(table_hbm.at[ids], bufs[b].at[pl.ds(k * 16, 16)],
                             gsems[b])

    def gather_wait(b):
        pltpu.make_async_copy(table_hbm.at[pl.ds(0, _CH)], bufs[b],
                              gsems[b]).wait()

    def out_window(j):
        return out_hbm.at[pl.ds(base + j * _CH, _CH), pl.ds(0, _D)]

    def scatter_start(j, b):
        pltpu.async_copy(bufs[b], out_window(j), ssems[b])

    def scatter_wait(j, b):
        pltpu.make_async_copy(bufs[b], out_window(j), ssems[b]).wait()

    for j in range(_LEAD):
        gather_start(j, j % _NBUF)

    def outer(g):
        for b in range(_NBUF):
            j = g + b
            gather_wait(b)
            buf = bufs[b]

            def row(i):
                for c in range(_D // 16):
                    sl = pl.ds(c * 16, 16)
                    buf[i, sl] = buf[i, sl] * _SCALE

            plsc.parallel_loop(0, _CH, 1, unroll=4)(row)

            scatter_start(j, b)
            jn = j + _LEAD
            bn = (b + _LEAD) % _NBUF

            @pl.when(jn < n_chunks)
            def _():
                @pl.when(jn >= _NBUF)
                def _():
                    scatter_wait(jn - _NBUF, bn)

                gather_start(jn, bn)

    pl.loop(0, n_chunks, step=_NBUF)(outer)

    for b in range(_NBUF):
        scatter_wait(n_chunks - _NBUF + b, b)


@jax.jit
def kernel(x, table):
    b, h = x.shape
    n = b * h
    assert n % (_NW * _CH * _NBUF) == 0
    n_chunks = n // (_NW * _CH)
    idx = x.reshape(n).astype(jnp.int32)

    mesh = plsc.VectorSubcoreMesh(core_axis_name="c", subcore_axis_name="s")
    out_padded = pl.kernel(
        functools.partial(_tec_body, n_chunks),
        out_type=jax.ShapeDtypeStruct((n, 2 * _D), jnp.float32),
        mesh=mesh,
        scratch_types=[
            pltpu.VMEM((n_chunks * _CH,), jnp.int32),
            [pltpu.VMEM((_CH, _D), jnp.float32) for _ in range(_NBUF)],
            [pltpu.SemaphoreType.DMA for _ in range(_NBUF)],
            [pltpu.SemaphoreType.DMA for _ in range(_NBUF)],
        ],
        compiler_params=pltpu.CompilerParams(use_tc_tiling_on_sc=False),
    )(idx, table)
    return out_padded[:, :_D].reshape(b, h, _D)
